# Initial kernel scaffold; baseline (speedup 1.0000x reference)
import jax
import jax.numpy as jnp
from jax.experimental import pallas as pl


def kernel(x, edge_index, edge_attr, graph_ids, W_node, b_node, W_edge, b_edge, W_msg, b_msg, W_upd, b_upd, W1, b1, a1, W2, b2, a2, W3, b3):
    raise NotImplementedError("write your pallas kernel here")


# speedup vs baseline: 1.2222x; 1.2222x over previous
import functools

import jax
import jax.numpy as jnp
from jax import lax
from jax.experimental import pallas as pl
from jax.experimental.pallas import tpu as pltpu
from jax.experimental.pallas import tpu_sc as plsc

_B = 256

_doth = functools.partial(lax.dot_general, precision=lax.Precision.HIGHEST)


def _mm(a, b):
    return lax.dot_general(a, b, dimension_numbers=(((a.ndim - 1,), (0,)), ((), ())))


def _embed_nodes_body(x_ref, w_ref, b_ref, o0, o1, o2, o3):
    h = jnp.maximum(_mm(x_ref[...], w_ref[...]) + b_ref[...], 0.0)
    o0[...] = h[:, 0:128]
    o1[...] = h[:, 128:256]
    o2[...] = h[:, 256:384]
    o3[...] = h[:, 384:512]


def _embed_nodes(x, W_node, b_node):
    N = x.shape[0]
    BN = 2000
    grid = N // BN
    q = jax.ShapeDtypeStruct((N, 128), jnp.float32)
    return pl.pallas_call(
        _embed_nodes_body,
        grid=(grid,),
        in_specs=[
            pl.BlockSpec((BN, x.shape[1]), lambda i: (i, 0)),
            pl.BlockSpec((x.shape[1], 512), lambda i: (0, 0)),
            pl.BlockSpec((1, 512), lambda i: (0, 0)),
        ],
        out_specs=[pl.BlockSpec((BN, 128), lambda i: (i, 0))] * 4,
        out_shape=[q, q, q, q],
    )(x, W_node, b_node.reshape(1, 512))


def _embed_edges(edge_attr, W_edge, b_edge):
    E = edge_attr.shape[0]
    BE = 2000
    grid = E // BE
    q = jax.ShapeDtypeStruct((E, 128), jnp.float32)
    return pl.pallas_call(
        _embed_nodes_body,
        grid=(grid,),
        in_specs=[
            pl.BlockSpec((BE, edge_attr.shape[1]), lambda i: (i, 0)),
            pl.BlockSpec((edge_attr.shape[1], 512), lambda i: (0, 0)),
            pl.BlockSpec((1, 512), lambda i: (0, 0)),
        ],
        out_specs=[pl.BlockSpec((BE, 128), lambda i: (i, 0))] * 4,
        out_shape=[q, q, q, q],
    )(edge_attr, W_edge, b_edge.reshape(1, 512))


def _update_body(h0, h1, h2, h3, p_ref, deg_ref, wm_ref, wu_ref, bm_ref,
                 bu_ref, o0, o1, o2, o3):
    hcat = jnp.concatenate([h0[...], h1[...], h2[...], h3[...]], axis=1)
    p = p_ref[...]
    wm = wm_ref[...].astype(jnp.bfloat16).astype(jnp.float32)
    agg = bm_ref[...] * (deg_ref[0, :, 0:1] + deg_ref[1, :, 0:1])
    rb = lambda v: v.astype(jnp.bfloat16).astype(jnp.float32)
    for qq in range(4):
        pq = p[qq] + p[4 + qq]
        wq = wm[qq * 128:(qq + 1) * 128, :]
        p1 = rb(pq)
        p2 = rb(pq - p1)
        p3 = rb(pq - p1 - p2)
        agg = agg + (_mm(p1, wq) + _mm(p2, wq) + _mm(p3, wq))
    hn = jnp.maximum(hcat + _mm(agg, wu_ref[...]) + bu_ref[...], 0.0)
    o0[...] = hn[:, 0:128]
    o1[...] = hn[:, 128:256]
    o2[...] = hn[:, 256:384]
    o3[...] = hn[:, 384:512]


def _update(hq, P8, deg2, Wm_t, Wu_t, bm_t, bu_t):
    N = hq[0].shape[0]
    BN = 2000
    grid = N // BN
    q = jax.ShapeDtypeStruct((N, 128), jnp.float32)
    hspec = pl.BlockSpec((BN, 128), lambda i: (i, 0))
    return pl.pallas_call(
        _update_body,
        grid=(grid,),
        in_specs=[hspec, hspec, hspec, hspec,
                  pl.BlockSpec((8, BN, 128), lambda i: (0, i, 0)),
                  pl.BlockSpec((2, BN, 16), lambda i: (0, i, 0)),
                  pl.BlockSpec((512, 512), lambda i: (0, 0)),
                  pl.BlockSpec((512, 512), lambda i: (0, 0)),
                  pl.BlockSpec((1, 512), lambda i: (0, 0)),
                  pl.BlockSpec((1, 512), lambda i: (0, 0))],
        out_specs=[hspec] * 4,
        out_shape=[q, q, q, q],
    )(*hq, P8, deg2, Wm_t, Wu_t, bm_t.reshape(1, 512), bu_t.reshape(1, 512))


def _readout_body(h0, h1, h2, h3, ids_ref, w1a, w1b, b1r, a1r, w2r, b2r,
                  a2r, w3r, b3r, mean_ref, logvar_ref, s_acc, c_acc):
    i = pl.program_id(0)
    n = pl.num_programs(0)

    @pl.when(i == 0)
    def _():
        s_acc[...] = jnp.zeros_like(s_acc)
        c_acc[...] = jnp.zeros_like(c_acc)

    hcat = jnp.concatenate([h0[...], h1[...], h2[...], h3[...]], axis=1)
    bn = hcat.shape[0]
    ids = ids_ref[...]
    iota = lax.broadcasted_iota(jnp.int32, (bn, _B), 1)
    onehot = (ids == iota).astype(jnp.float32)
    s_acc[...] += _doth(onehot, hcat, (((0,), (0,)), ((), ())))
    c_acc[...] += _doth(onehot, jnp.ones((bn, 128), jnp.float32),
                        (((0,), (0,)), ((), ())))

    @pl.when(i == n - 1)
    def _():
        s = s_acc[...]
        cnt = jnp.maximum(c_acc[...][:, 0:1], 1.0)
        m = s / cnt
        z1 = _mm(s, w1a[...]) + _mm(m, w1b[...]) + b1r[...]
        h1v = jnp.where(z1 >= 0, z1, a1r[...] * z1)
        z2 = _mm(h1v, w2r[...]) + b2r[...]
        h2v = jnp.where(z2 >= 0, z2, a2r[...] * z2)
        out = _mm(h2v, w3r[...]) + b3r[...]
        mean_ref[...] = out[:, 0:1]
        logvar_ref[...] = out[:, 1:2]


def _readout(hq, gid2, W1, b1, a1, W2, b2, a2, W3, b3):
    N = hq[0].shape[0]
    BN = 2000
    grid = N // BN
    W3p = jnp.zeros((512, 128), jnp.float32).at[:, :2].set(W3)
    b3p = jnp.zeros((1, 128), jnp.float32).at[:, :2].set(b3.reshape(1, 2))
    hspec = pl.BlockSpec((BN, 128), lambda i: (i, 0))
    full = lambda s: pl.BlockSpec(s, lambda i: tuple(0 for _ in s))
    return pl.pallas_call(
        _readout_body,
        grid=(grid,),
        in_specs=[hspec, hspec, hspec, hspec,
                  pl.BlockSpec((BN, 1), lambda i: (i, 0)),
                  full((512, 512)), full((512, 512)), full((1, 512)),
                  full((1, 1)), full((512, 512)), full((1, 512)),
                  full((1, 1)), full((512, 128)), full((1, 128))],
        out_specs=[full((_B, 1)), full((_B, 1))],
        out_shape=[jax.ShapeDtypeStruct((_B, 1), jnp.float32),
                   jax.ShapeDtypeStruct((_B, 1), jnp.float32)],
        scratch_shapes=[pltpu.VMEM((_B, 512), jnp.float32),
                        pltpu.VMEM((_B, 128), jnp.float32)],
    )(*hq, gid2, W1[:512], W1[512:], b1.reshape(1, 512),
      a1.reshape(1, 1), W2, b2.reshape(1, 512), a2.reshape(1, 1), W3p, b3p)


_CH = 40


def _sc_scatter_body(NP, E, h0, h1, h2, h3, e0, e1, e2, e3, src_hbm,
                     dst_hbm, zrow_hbm, p_out, sidx, didx, hrows, erows,
                     prod, acc, sem):
    c = lax.axis_index("c")
    s = lax.axis_index("s")
    wid = c * 16 + s
    ep = E // 32
    base = wid * ep
    nrows = NP // 16
    htabs = (h0, h1, h2, h3)
    etabs = (e0, e1, e2, e3)
    for q in range(4):
        ht = htabs[q]
        et = etabs[q]
        pltpu.sync_copy(zrow_hbm, acc.at[pl.ds(s * nrows, nrows)])
        plsc.subcore_barrier()

        def chunk(ck, carry, ht=ht, et=et):
            eb = base + ck * _CH
            pltpu.sync_copy(src_hbm.at[pl.ds(eb, _CH)], sidx)
            pltpu.sync_copy(dst_hbm.at[pl.ds(eb, _CH)], didx)
            pltpu.async_copy(ht.at[sidx], hrows, sem).wait()
            pltpu.sync_copy(et.at[pl.ds(eb, _CH), :], erows)

            def mul(e, carry2):
                for f in range(8):
                    sl = pl.ds(f * 16, 16)
                    v = hrows[e, sl] * erows[e, sl]
                    prod[e, sl] = v.astype(jnp.bfloat16).astype(jnp.float32)
                return carry2

            lax.fori_loop(0, _CH, mul, 0, unroll=False)
            pltpu.sync_copy(prod, acc.at[didx], add=True)
            return carry

        lax.fori_loop(0, ep // _CH, chunk, 0, unroll=False)
        plsc.subcore_barrier()
        row0 = (c * 4 + q) * NP + s * nrows
        pltpu.sync_copy(acc.at[pl.ds(s * nrows, nrows)],
                        p_out.at[pl.ds(row0, nrows)])


def _scatter_stage(hq, ehq, src, dst, zrow, NP):
    E = src.shape[0]
    mesh = plsc.VectorSubcoreMesh(core_axis_name="c", subcore_axis_name="s")
    body = functools.partial(_sc_scatter_body, NP, E)
    f = pl.kernel(
        body,
        out_type=jax.ShapeDtypeStruct((8 * NP, 128), jnp.float32),
        mesh=mesh,
        scratch_types=[
            pltpu.VMEM((_CH,), jnp.int32),
            pltpu.VMEM((_CH,), jnp.int32),
            pltpu.VMEM((_CH, 128), jnp.float32),
            pltpu.VMEM((_CH, 128), jnp.float32),
            pltpu.VMEM((_CH, 128), jnp.float32),
            pltpu.VMEM_SHARED((NP, 128), jnp.float32),
            pltpu.SemaphoreType.DMA,
        ],
    )
    P8 = f(*hq, *ehq, src, dst, zrow)
    return P8.reshape(8, NP, 128)


def _sc_deg_body(NP, E, dst_hbm, ones_hbm, z16_hbm, deg_out, didx, ones_v,
                 acc16):
    c = lax.axis_index("c")
    s = lax.axis_index("s")
    wid = c * 16 + s
    ep = E // 32
    base = wid * ep
    nrows = NP // 16
    pltpu.sync_copy(z16_hbm, acc16.at[pl.ds(s * nrows, nrows)])
    pltpu.sync_copy(ones_hbm, ones_v)
    plsc.subcore_barrier()

    def chunk(ck, carry):
        eb = base + ck * _CH
        pltpu.sync_copy(dst_hbm.at[pl.ds(eb, _CH)], didx)
        pltpu.sync_copy(ones_v, acc16.at[didx], add=True)
        return carry

    lax.fori_loop(0, ep // _CH, chunk, 0, unroll=False)
    plsc.subcore_barrier()
    pltpu.sync_copy(acc16.at[pl.ds(s * nrows, nrows)],
                    deg_out.at[pl.ds(c * NP + s * nrows, nrows)])


def _deg_stage(dst, ones16, z16, NP):
    E = dst.shape[0]
    mesh = plsc.VectorSubcoreMesh(core_axis_name="c", subcore_axis_name="s")
    body = functools.partial(_sc_deg_body, NP, E)
    f = pl.kernel(
        body,
        out_type=jax.ShapeDtypeStruct((2 * NP, 16), jnp.float32),
        mesh=mesh,
        scratch_types=[
            pltpu.VMEM((_CH,), jnp.int32),
            pltpu.VMEM((_CH, 16), jnp.float32),
            pltpu.VMEM_SHARED((NP, 16), jnp.float32),
        ],
    )
    return f(dst, ones16, z16).reshape(2, NP, 16)


def kernel(x, edge_index, edge_attr, graph_ids, W_node, b_node, W_edge,
           b_edge, W_msg, b_msg, W_upd, b_upd, W1, b1, a1, W2, b2, a2,
           W3, b3):
    N = x.shape[0]
    L = W_msg.shape[0]
    src = edge_index[0]
    dst = edge_index[1]

    hq = _embed_nodes(x, W_node, b_node)
    ehq = _embed_edges(edge_attr, W_edge, b_edge)
    NP = ((N + 127) // 128) * 128
    nrows = NP // 16
    zrow = jnp.zeros((nrows, 128), jnp.float32)
    z16 = jnp.zeros((nrows, 16), jnp.float32)
    ones16 = jnp.ones((_CH, 16), jnp.float32)
    deg2 = _deg_stage(dst, ones16, z16, NP)

    for t in range(L):
        P8 = _scatter_stage(hq, ehq, src, dst, zrow, NP)
        hq = _update(hq, P8, deg2, W_msg[t], W_upd[t], b_msg[t], b_upd[t])

    gid2 = graph_ids.reshape(N, 1)
    mean, logvar = _readout(hq, gid2, W1, b1, a1, W2, b2, a2, W3, b3)
    return (mean, logvar)


# speedup vs baseline: 2.2639x; 1.8523x over previous
import functools

import jax
import jax.numpy as jnp
from jax import lax
from jax.experimental import pallas as pl
from jax.experimental.pallas import tpu as pltpu
from jax.experimental.pallas import tpu_sc as plsc

_B = 256

_doth = functools.partial(lax.dot_general, precision=lax.Precision.HIGHEST)


def _mm(a, b):
    return lax.dot_general(a, b, dimension_numbers=(((a.ndim - 1,), (0,)), ((), ())))


def _embed_nodes_body(x_ref, w_ref, b_ref, o0, o1, o2, o3):
    h = jnp.maximum(_mm(x_ref[...], w_ref[...]) + b_ref[...], 0.0)
    o0[...] = h[:, 0:128]
    o1[...] = h[:, 128:256]
    o2[...] = h[:, 256:384]
    o3[...] = h[:, 384:512]


def _embed_nodes(x, W_node, b_node):
    N = x.shape[0]
    BN = 2000
    grid = N // BN
    q = jax.ShapeDtypeStruct((N, 128), jnp.float32)
    return pl.pallas_call(
        _embed_nodes_body,
        grid=(grid,),
        in_specs=[
            pl.BlockSpec((BN, x.shape[1]), lambda i: (i, 0)),
            pl.BlockSpec((x.shape[1], 512), lambda i: (0, 0)),
            pl.BlockSpec((1, 512), lambda i: (0, 0)),
        ],
        out_specs=[pl.BlockSpec((BN, 128), lambda i: (i, 0))] * 4,
        out_shape=[q, q, q, q],
    )(x, W_node, b_node.reshape(1, 512))


def _embed_edges(edge_attr, W_edge, b_edge):
    E = edge_attr.shape[0]
    BE = 2000
    grid = E // BE
    q = jax.ShapeDtypeStruct((E, 128), jnp.float32)
    return pl.pallas_call(
        _embed_nodes_body,
        grid=(grid,),
        in_specs=[
            pl.BlockSpec((BE, edge_attr.shape[1]), lambda i: (i, 0)),
            pl.BlockSpec((edge_attr.shape[1], 512), lambda i: (0, 0)),
            pl.BlockSpec((1, 512), lambda i: (0, 0)),
        ],
        out_specs=[pl.BlockSpec((BE, 128), lambda i: (i, 0))] * 4,
        out_shape=[q, q, q, q],
    )(edge_attr, W_edge, b_edge.reshape(1, 512))


def _update_body(h0, h1, h2, h3, p_ref, deg_ref, wm_ref, wu_ref, bm_ref,
                 bu_ref, o0, o1, o2, o3):
    hcat = jnp.concatenate([h0[...], h1[...], h2[...], h3[...]], axis=1)
    p = p_ref[...]
    wm = wm_ref[...].astype(jnp.bfloat16).astype(jnp.float32)
    agg = bm_ref[...] * (deg_ref[0, :, 0:1] + deg_ref[1, :, 0:1])
    rb = lambda v: v.astype(jnp.bfloat16).astype(jnp.float32)
    for qq in range(4):
        pq = p[qq] + p[4 + qq]
        wq = wm[qq * 128:(qq + 1) * 128, :]
        p1 = rb(pq)
        p2 = rb(pq - p1)
        p3 = rb(pq - p1 - p2)
        agg = agg + (_mm(p1, wq) + _mm(p2, wq) + _mm(p3, wq))
    hn = jnp.maximum(hcat + _mm(agg, wu_ref[...]) + bu_ref[...], 0.0)
    o0[...] = hn[:, 0:128]
    o1[...] = hn[:, 128:256]
    o2[...] = hn[:, 256:384]
    o3[...] = hn[:, 384:512]


def _update(hq, P8, deg2, Wm_t, Wu_t, bm_t, bu_t):
    N = hq[0].shape[0]
    BN = 2000
    grid = N // BN
    q = jax.ShapeDtypeStruct((N, 128), jnp.float32)
    hspec = pl.BlockSpec((BN, 128), lambda i: (i, 0))
    return pl.pallas_call(
        _update_body,
        grid=(grid,),
        in_specs=[hspec, hspec, hspec, hspec,
                  pl.BlockSpec((8, BN, 128), lambda i: (0, i, 0)),
                  pl.BlockSpec((2, BN, 16), lambda i: (0, i, 0)),
                  pl.BlockSpec((512, 512), lambda i: (0, 0)),
                  pl.BlockSpec((512, 512), lambda i: (0, 0)),
                  pl.BlockSpec((1, 512), lambda i: (0, 0)),
                  pl.BlockSpec((1, 512), lambda i: (0, 0))],
        out_specs=[hspec] * 4,
        out_shape=[q, q, q, q],
    )(*hq, P8, deg2, Wm_t, Wu_t, bm_t.reshape(1, 512), bu_t.reshape(1, 512))


def _readout_body(h0, h1, h2, h3, ids_ref, w1a, w1b, b1r, a1r, w2r, b2r,
                  a2r, w3r, b3r, mean_ref, logvar_ref, s_acc, c_acc):
    i = pl.program_id(0)
    n = pl.num_programs(0)

    @pl.when(i == 0)
    def _():
        s_acc[...] = jnp.zeros_like(s_acc)
        c_acc[...] = jnp.zeros_like(c_acc)

    hcat = jnp.concatenate([h0[...], h1[...], h2[...], h3[...]], axis=1)
    bn = hcat.shape[0]
    ids = ids_ref[...]
    iota = lax.broadcasted_iota(jnp.int32, (bn, _B), 1)
    onehot = (ids == iota).astype(jnp.float32)
    s_acc[...] += _doth(onehot, hcat, (((0,), (0,)), ((), ())))
    c_acc[...] += _doth(onehot, jnp.ones((bn, 128), jnp.float32),
                        (((0,), (0,)), ((), ())))

    @pl.when(i == n - 1)
    def _():
        s = s_acc[...]
        cnt = jnp.maximum(c_acc[...][:, 0:1], 1.0)
        m = s / cnt
        z1 = _mm(s, w1a[...]) + _mm(m, w1b[...]) + b1r[...]
        h1v = jnp.where(z1 >= 0, z1, a1r[...] * z1)
        z2 = _mm(h1v, w2r[...]) + b2r[...]
        h2v = jnp.where(z2 >= 0, z2, a2r[...] * z2)
        out = _mm(h2v, w3r[...]) + b3r[...]
        mean_ref[...] = out[:, 0:1]
        logvar_ref[...] = out[:, 1:2]


def _readout(hq, gid2, W1, b1, a1, W2, b2, a2, W3, b3):
    N = hq[0].shape[0]
    BN = 2000
    grid = N // BN
    W3p = jnp.zeros((512, 128), jnp.float32).at[:, :2].set(W3)
    b3p = jnp.zeros((1, 128), jnp.float32).at[:, :2].set(b3.reshape(1, 2))
    hspec = pl.BlockSpec((BN, 128), lambda i: (i, 0))
    full = lambda s: pl.BlockSpec(s, lambda i: tuple(0 for _ in s))
    return pl.pallas_call(
        _readout_body,
        grid=(grid,),
        in_specs=[hspec, hspec, hspec, hspec,
                  pl.BlockSpec((BN, 1), lambda i: (i, 0)),
                  full((512, 512)), full((512, 512)), full((1, 512)),
                  full((1, 1)), full((512, 512)), full((1, 512)),
                  full((1, 1)), full((512, 128)), full((1, 128))],
        out_specs=[full((_B, 1)), full((_B, 1))],
        out_shape=[jax.ShapeDtypeStruct((_B, 1), jnp.float32),
                   jax.ShapeDtypeStruct((_B, 1), jnp.float32)],
        scratch_shapes=[pltpu.VMEM((_B, 512), jnp.float32),
                        pltpu.VMEM((_B, 128), jnp.float32)],
    )(*hq, gid2, W1[:512], W1[512:], b1.reshape(1, 512),
      a1.reshape(1, 1), W2, b2.reshape(1, 512), a2.reshape(1, 1), W3p, b3p)


_CH = 40


def _sc_scatter_body(NP, E, h0, h1, h2, h3, e0, e1, e2, e3, src_hbm,
                     dst_hbm, zrow_hbm, p_out, si0, si1, di0, di1, hr0,
                     hr1, er0, er1, pr, acc, gs0, gs1):
    c = lax.axis_index("c")
    s = lax.axis_index("s")
    wid = c * 16 + s
    ep = E // 32
    base = wid * ep
    nmain = ep // _CH
    nrows = NP // 16
    htabs = (h0, h1, h2, h3)
    etabs = (e0, e1, e2, e3)
    sib = (si0, si1)
    dib = (di0, di1)
    hrb = (hr0, hr1)
    erb = (er0, er1)
    gsb = (gs0, gs1)
    for q in range(4):
        ht = htabs[q]
        et = etabs[q]
        pltpu.sync_copy(zrow_hbm, acc.at[pl.ds(s * nrows, nrows)])
        plsc.subcore_barrier()

        def issue(ck, b, ht=ht, et=et):
            eb = base + ck * _CH
            pltpu.sync_copy(src_hbm.at[pl.ds(eb, _CH)], sib[b])
            pltpu.sync_copy(dst_hbm.at[pl.ds(eb, _CH)], dib[b])
            pltpu.async_copy(ht.at[sib[b]], hrb[b], gsb[b])
            pltpu.async_copy(et.at[pl.ds(eb, _CH), :], erb[b], gsb[b])

        def consume(b, et=et):
            pltpu.make_async_copy(et.at[pl.ds(0, _CH), :], hrb[b],
                                  gsb[b]).wait()
            pltpu.make_async_copy(et.at[pl.ds(0, _CH), :], erb[b],
                                  gsb[b]).wait()

            def mul(e, carry):
                for f in range(8):
                    sl = pl.ds(f * 16, 16)
                    v = hrb[b][e, sl] * erb[b][e, sl]
                    pr[e, sl] = v.astype(jnp.bfloat16).astype(jnp.float32)
                return carry

            lax.fori_loop(0, _CH, mul, 0, unroll=False)
            pltpu.sync_copy(pr, acc.at[dib[b]], add=True)

        issue(0, 0)

        def step(jj, carry):
            issue(2 * jj + 1, 1)
            consume(0)
            issue(2 * jj + 2, 0)
            consume(1)
            return carry

        lax.fori_loop(0, (nmain - 1) // 2, step, 0, unroll=False)
        consume(0)

        plsc.subcore_barrier()
        row0 = (c * 4 + q) * NP + s * nrows
        pltpu.sync_copy(acc.at[pl.ds(s * nrows, nrows)],
                        p_out.at[pl.ds(row0, nrows)])


def _scatter_stage(hq, ehq, src, dst, zrow, NP):
    E = src.shape[0]
    mesh = plsc.VectorSubcoreMesh(core_axis_name="c", subcore_axis_name="s")
    body = functools.partial(_sc_scatter_body, NP, E)
    f = pl.kernel(
        body,
        out_type=jax.ShapeDtypeStruct((8 * NP, 128), jnp.float32),
        mesh=mesh,
        scratch_types=[
            pltpu.VMEM((_CH,), jnp.int32),
            pltpu.VMEM((_CH,), jnp.int32),
            pltpu.VMEM((_CH,), jnp.int32),
            pltpu.VMEM((_CH,), jnp.int32),
            pltpu.VMEM((_CH, 128), jnp.float32),
            pltpu.VMEM((_CH, 128), jnp.float32),
            pltpu.VMEM((_CH, 128), jnp.float32),
            pltpu.VMEM((_CH, 128), jnp.float32),
            pltpu.VMEM((_CH, 128), jnp.float32),
            pltpu.VMEM_SHARED((NP, 128), jnp.float32),
            pltpu.SemaphoreType.DMA,
            pltpu.SemaphoreType.DMA,
        ],
    )
    P8 = f(*hq, *ehq, src, dst, zrow)
    return P8.reshape(8, NP, 128)


_DCH = 40


def _sc_deg_body(NP, E, dst_hbm, ones_hbm, z16_hbm, deg_out, didx, ones_v,
                 acc16):
    c = lax.axis_index("c")
    s = lax.axis_index("s")
    wid = c * 16 + s
    ep = E // 32
    base = wid * ep
    nrows = NP // 16
    pltpu.sync_copy(z16_hbm, acc16.at[pl.ds(s * nrows, nrows)])
    pltpu.sync_copy(ones_hbm, ones_v)
    plsc.subcore_barrier()

    def chunk(ck, carry):
        eb = base + ck * _DCH
        pltpu.sync_copy(dst_hbm.at[pl.ds(eb, _DCH)], didx)
        pltpu.sync_copy(ones_v, acc16.at[didx], add=True)
        return carry

    lax.fori_loop(0, ep // _DCH, chunk, 0, unroll=False)
    plsc.subcore_barrier()
    pltpu.sync_copy(acc16.at[pl.ds(s * nrows, nrows)],
                    deg_out.at[pl.ds(c * NP + s * nrows, nrows)])


def _deg_stage(dst, ones16, z16, NP):
    E = dst.shape[0]
    mesh = plsc.VectorSubcoreMesh(core_axis_name="c", subcore_axis_name="s")
    body = functools.partial(_sc_deg_body, NP, E)
    f = pl.kernel(
        body,
        out_type=jax.ShapeDtypeStruct((2 * NP, 16), jnp.float32),
        mesh=mesh,
        scratch_types=[
            pltpu.VMEM((_DCH,), jnp.int32),
            pltpu.VMEM((_DCH, 16), jnp.float32),
            pltpu.VMEM_SHARED((NP, 16), jnp.float32),
        ],
    )
    return f(dst, ones16, z16).reshape(2, NP, 16)


def kernel(x, edge_index, edge_attr, graph_ids, W_node, b_node, W_edge,
           b_edge, W_msg, b_msg, W_upd, b_upd, W1, b1, a1, W2, b2, a2,
           W3, b3):
    N = x.shape[0]
    L = W_msg.shape[0]
    src = edge_index[0]
    dst = edge_index[1]

    hq = _embed_nodes(x, W_node, b_node)
    ehq = _embed_edges(edge_attr, W_edge, b_edge)
    NP = ((N + 127) // 128) * 128
    nrows = NP // 16
    zrow = jnp.zeros((nrows, 128), jnp.float32)
    z16 = jnp.zeros((nrows, 16), jnp.float32)
    ones16 = jnp.ones((_DCH, 16), jnp.float32)
    deg2 = _deg_stage(dst, ones16, z16, NP)

    for t in range(L):
        P8 = _scatter_stage(hq, ehq, src, dst, zrow, NP)
        hq = _update(hq, P8, deg2, W_msg[t], W_upd[t], b_msg[t], b_upd[t])

    gid2 = graph_ids.reshape(N, 1)
    mean, logvar = _readout(hq, gid2, W1, b1, a1, W2, b2, a2, W3, b3)
    return (mean, logvar)
